# baseline (speedup 1.0000x reference)
import functools

import jax
import jax.numpy as jnp
from jax import lax
from jax.experimental import pallas as pl
from jax.experimental.pallas import tpu as pltpu
from jax.experimental.pallas import tpu_sc as plsc

TOPK = 8
NE = 64
H = 1024
NC = 2
NS = 16
NW = NC * NS
CHUNKS = 1


TC_FAN = 4


def _logits_kernel(x_ref, w_ref, out_ref):
    w = w_ref[...]
    tpc = out_ref.shape[2]
    for j in range(TC_FAN):
        out_ref[j] = lax.dot_general(
            w, x_ref[pl.ds(j * tpc, tpc), :], (((1,), (1,)), ((), ())),
            preferred_element_type=jnp.float32,
        )


def _tc_logits_chunk(x, weight, tpc, chunk):
    return pl.pallas_call(
        _logits_kernel,
        grid=(NW // TC_FAN,),
        in_specs=[
            pl.BlockSpec((TC_FAN * tpc, H), lambda i, c=chunk: (c * NW // TC_FAN + i, 0)),
            pl.BlockSpec((NE, H), lambda i: (0, 0)),
        ],
        out_specs=pl.BlockSpec((TC_FAN, NE, tpc), lambda i: (i, 0, 0)),
        out_shape=jax.ShapeDtypeStruct((NW, NE, tpc), jnp.float32),
        cost_estimate=pl.CostEstimate(
            flops=2 * NW * tpc * H * NE,
            transcendentals=0,
            bytes_accessed=NW * tpc * H * 4 + NE * H * 4 + NW * NE * tpc * 4,
        ),
    )(x, weight)


def _make_sc_topk_body(tpc):
    def _sc_topk_body(lg_hbm, idx_hbm, wgt_hbm, lg_v, idx_v, wgt_v):
        wid = lax.axis_index("s") * NC + lax.axis_index("c")
        pltpu.sync_copy(lg_hbm.at[wid], lg_v)

        def group(g, carry):
            t0 = pl.multiple_of(g * 16, 16)
            neg = jnp.full((16,), -jnp.inf, jnp.float32)
            zero = jnp.zeros((16,), jnp.int32)
            vs = [neg] * TOPK
            ixs = [zero] * TOPK
            for e in range(NE):
                x = lg_v[e, pl.ds(t0, 16)]
                ev = jnp.full((16,), e, jnp.int32)
                gt = [x > vs[j] for j in range(TOPK)]
                nv = [None] * TOPK
                ni = [None] * TOPK
                nv[0] = jnp.where(gt[0], x, vs[0])
                ni[0] = jnp.where(gt[0], ev, ixs[0])
                for j in range(1, TOPK):
                    nv[j] = jnp.where(gt[j], jnp.where(gt[j - 1], vs[j - 1], x), vs[j])
                    ni[j] = jnp.where(gt[j], jnp.where(gt[j - 1], ixs[j - 1], ev), ixs[j])
                vs, ixs = nv, ni
            m = vs[0]
            es = [jnp.exp(vs[k] - m) for k in range(TOPK)]
            s = es[0]
            for k in range(1, TOPK):
                s = s + es[k]
            r = 1.0 / s
            for k in range(TOPK):
                idx_v[k, pl.ds(t0, 16)] = ixs[k]
                wgt_v[k, pl.ds(t0, 16)] = es[k] * r
            return carry

        lax.fori_loop(0, tpc // 16, group, 0)
        pltpu.sync_copy(idx_v, idx_hbm.at[wid])
        pltpu.sync_copy(wgt_v, wgt_hbm.at[wid])

    return _sc_topk_body


def _sc_topk_chunk(logits3, tpc):
    mesh = plsc.VectorSubcoreMesh(core_axis_name="c", subcore_axis_name="s")
    fn = pl.kernel(
        _make_sc_topk_body(tpc),
        out_type=[
            jax.ShapeDtypeStruct((NW, TOPK, tpc), jnp.int32),
            jax.ShapeDtypeStruct((NW, TOPK, tpc), jnp.float32),
        ],
        mesh=mesh,
        scratch_types=[
            pltpu.VMEM((NE, tpc), jnp.float32),
            pltpu.VMEM((TOPK, tpc), jnp.int32),
            pltpu.VMEM((TOPK, tpc), jnp.float32),
        ],
        cost_estimate=pl.CostEstimate(
            flops=50 * NW * NE * tpc,
            transcendentals=NW * TOPK * tpc,
            bytes_accessed=NW * NE * tpc * 4 + 2 * NW * TOPK * tpc * 4,
        ),
    )
    return fn(logits3)


def kernel(hidden_states, weight):
    x = hidden_states.reshape(-1, hidden_states.shape[-1])
    t = x.shape[0]
    tpc = t // (CHUNKS * NW)
    idx_parts = [None] * CHUNKS
    wgt_parts = [None] * CHUNKS
    logits_parts = [None] * CHUNKS
    logits_parts[0] = _tc_logits_chunk(x, weight, tpc, 0)
    for c in range(1, CHUNKS):
        logits_parts[c] = _tc_logits_chunk(x, weight, tpc, c)
        idx_parts[c - 1], wgt_parts[c - 1] = _sc_topk_chunk(logits_parts[c - 1], tpc)
    idx_parts[-1], wgt_parts[-1] = _sc_topk_chunk(logits_parts[-1], tpc)
    idx3 = jnp.concatenate(idx_parts, axis=0)
    wgt3 = jnp.concatenate(wgt_parts, axis=0)
    idx = idx3.transpose(0, 2, 1).reshape(t, TOPK)
    wgt = wgt3.transpose(0, 2, 1).reshape(t, TOPK)
    return idx, wgt


# speedup vs baseline: 1.0012x; 1.0012x over previous
import functools

import jax
import jax.numpy as jnp
from jax import lax
from jax.experimental import pallas as pl
from jax.experimental.pallas import tpu as pltpu
from jax.experimental.pallas import tpu_sc as plsc

TOPK = 8
NE = 64
H = 1024
NC = 2
NS = 16
NW = NC * NS
CHUNKS = 1


TC_FAN = 2


def _logits_kernel(x_ref, w_ref, out_ref):
    w = w_ref[...]
    tpc = out_ref.shape[2]
    for j in range(TC_FAN):
        out_ref[j] = lax.dot_general(
            w, x_ref[pl.ds(j * tpc, tpc), :], (((1,), (1,)), ((), ())),
            preferred_element_type=jnp.float32,
        )


def _tc_logits_chunk(x, weight, tpc, chunk):
    return pl.pallas_call(
        _logits_kernel,
        grid=(NW // TC_FAN,),
        in_specs=[
            pl.BlockSpec((TC_FAN * tpc, H), lambda i, c=chunk: (c * NW // TC_FAN + i, 0)),
            pl.BlockSpec((NE, H), lambda i: (0, 0)),
        ],
        out_specs=pl.BlockSpec((TC_FAN, NE, tpc), lambda i: (i, 0, 0)),
        out_shape=jax.ShapeDtypeStruct((NW, NE, tpc), jnp.float32),
        cost_estimate=pl.CostEstimate(
            flops=2 * NW * tpc * H * NE,
            transcendentals=0,
            bytes_accessed=NW * tpc * H * 4 + NE * H * 4 + NW * NE * tpc * 4,
        ),
    )(x, weight)


def _make_sc_topk_body(tpc, t):
    half = tpc // 2

    def _sc_topk_body(lg_hbm, idx_hbm, wgt_hbm, lg_a, lg_b, idxw_v, wgtw_v,
                      sem_a, sem_b):
        wid = lax.axis_index("s") * NC + lax.axis_index("c")
        cp_a = pltpu.make_async_copy(lg_hbm.at[wid, :, pl.ds(0, half)], lg_a, sem_a)
        cp_b = pltpu.make_async_copy(lg_hbm.at[wid, :, pl.ds(half, half)], lg_b, sem_b)
        cp_a.start()
        cp_b.start()

        def make_group(lg_v, base):
            def group(g, carry):
                t0 = pl.multiple_of(g * 16, 16)
                neg = jnp.full((16,), -jnp.inf, jnp.float32)
                zero = jnp.zeros((16,), jnp.int32)
                vs = [neg] * TOPK
                ixs = [zero] * TOPK
                for e in range(NE):
                    x = lg_v[e, pl.ds(t0, 16)]
                    ev = jnp.full((16,), e, jnp.int32)
                    gt = [x > vs[j] for j in range(TOPK)]
                    nv = [None] * TOPK
                    ni = [None] * TOPK
                    nv[0] = jnp.where(gt[0], x, vs[0])
                    ni[0] = jnp.where(gt[0], ev, ixs[0])
                    for j in range(1, TOPK):
                        nv[j] = jnp.where(gt[j], jnp.where(gt[j - 1], vs[j - 1], x), vs[j])
                        ni[j] = jnp.where(gt[j], jnp.where(gt[j - 1], ixs[j - 1], ev), ixs[j])
                    vs, ixs = nv, ni
                m = vs[0]
                es = [jnp.full((16,), 1.0, jnp.float32)]
                for k in range(1, TOPK):
                    es.append(jnp.exp(vs[k] - m))
                s = es[0]
                for k in range(1, TOPK):
                    s = s + es[k]
                r = 1.0 / s
                for k in range(TOPK):
                    idxw_v[k, pl.ds(base + t0, 16)] = ixs[k]
                    wgtw_v[k, pl.ds(base + t0, 16)] = es[k] * r
                return carry

            return group

        cp_a.wait()
        lax.fori_loop(0, half // 16, make_group(lg_a, 0), 0)
        cp_b.wait()
        lax.fori_loop(0, half // 16, make_group(lg_b, half), 0)
        pltpu.sync_copy(idxw_v, idx_hbm.at[wid])
        pltpu.sync_copy(wgtw_v, wgt_hbm.at[wid])

    return _sc_topk_body


def _sc_topk_chunk(logits3, tpc, t):
    mesh = plsc.VectorSubcoreMesh(core_axis_name="c", subcore_axis_name="s")
    half = tpc // 2
    fn = pl.kernel(
        _make_sc_topk_body(tpc, t),
        out_type=[
            jax.ShapeDtypeStruct((NW, TOPK, tpc), jnp.int32),
            jax.ShapeDtypeStruct((NW, TOPK, tpc), jnp.float32),
        ],
        mesh=mesh,
        scratch_types=[
            pltpu.VMEM((NE, half), jnp.float32),
            pltpu.VMEM((NE, half), jnp.float32),
            pltpu.VMEM((TOPK, tpc), jnp.int32),
            pltpu.VMEM((TOPK, tpc), jnp.float32),
            pltpu.SemaphoreType.DMA,
            pltpu.SemaphoreType.DMA,
        ],
        cost_estimate=pl.CostEstimate(
            flops=50 * NW * NE * tpc,
            transcendentals=NW * TOPK * tpc,
            bytes_accessed=NW * NE * tpc * 4 + 2 * NW * TOPK * tpc * 4,
        ),
    )
    return fn(logits3)


def kernel(hidden_states, weight):
    x = hidden_states.reshape(-1, hidden_states.shape[-1])
    t = x.shape[0]
    tpc = t // (CHUNKS * NW)
    logits3 = _tc_logits_chunk(x, weight, tpc, 0)
    idx3, wgt3 = _sc_topk_chunk(logits3, tpc, t)
    idx = idx3.transpose(0, 2, 1).reshape(t, TOPK)
    wgt = wgt3.transpose(0, 2, 1).reshape(t, TOPK)
    return idx, wgt


# speedup vs baseline: 1.0169x; 1.0157x over previous
import functools

import jax
import jax.numpy as jnp
from jax import lax
from jax.experimental import pallas as pl
from jax.experimental.pallas import tpu as pltpu
from jax.experimental.pallas import tpu_sc as plsc

TOPK = 8
NE = 64
H = 1024
NC = 2
NS = 16
NW = NC * NS
CHUNKS = 1


TC_FAN = 2


def _logits_kernel(x_ref, w_ref, out_ref):
    w = w_ref[...]
    half = out_ref.shape[2]
    for j in range(out_ref.shape[0]):
        out_ref[j] = lax.dot_general(
            w, x_ref[pl.ds(j * half, half), :], (((1,), (1,)), ((), ())),
            preferred_element_type=jnp.float32,
        )


def _tc_logits_chunk(x, weight, tpc, chunk):
    half = tpc // 2
    nslab = 2 * TC_FAN
    return pl.pallas_call(
        _logits_kernel,
        grid=(NW // TC_FAN,),
        in_specs=[
            pl.BlockSpec((TC_FAN * tpc, H), lambda i, c=chunk: (c * NW // TC_FAN + i, 0)),
            pl.BlockSpec((NE, H), lambda i: (0, 0)),
        ],
        out_specs=pl.BlockSpec((nslab, NE, half), lambda i: (i, 0, 0)),
        out_shape=jax.ShapeDtypeStruct((2 * NW, NE, half), jnp.float32),
        cost_estimate=pl.CostEstimate(
            flops=2 * NW * tpc * H * NE,
            transcendentals=0,
            bytes_accessed=NW * tpc * H * 4 + NE * H * 4 + NW * NE * tpc * 4,
        ),
    )(x, weight)


def _make_sc_topk_body(tpc, t):
    half = tpc // 2

    def _sc_topk_body(lg_hbm, idx_hbm, wgt_hbm, lg_a, lg_b, idxw_v, wgtw_v,
                      sem_a, sem_b):
        wid = lax.axis_index("s") * NC + lax.axis_index("c")
        cp_a = pltpu.make_async_copy(lg_hbm.at[2 * wid], lg_a, sem_a)
        cp_b = pltpu.make_async_copy(lg_hbm.at[2 * wid + 1], lg_b, sem_b)
        cp_a.start()
        cp_b.start()

        def make_group(lg_v, base):
            def group(g, carry):
                t0 = pl.multiple_of(g * 16, 16)
                neg = jnp.full((16,), -jnp.inf, jnp.float32)
                zero = jnp.zeros((16,), jnp.int32)
                vs = [neg] * TOPK
                ixs = [zero] * TOPK
                for e in range(NE):
                    x = lg_v[e, pl.ds(t0, 16)]
                    ev = jnp.full((16,), e, jnp.int32)
                    gt = [x > vs[j] for j in range(TOPK)]
                    nv = [None] * TOPK
                    ni = [None] * TOPK
                    nv[0] = jnp.maximum(x, vs[0])
                    ni[0] = jnp.where(gt[0], ev, ixs[0])
                    for j in range(1, TOPK):
                        nv[j] = jnp.maximum(jnp.minimum(x, vs[j - 1]), vs[j])
                        ni[j] = jnp.where(gt[j], jnp.where(gt[j - 1], ixs[j - 1], ev), ixs[j])
                    vs, ixs = nv, ni
                m = vs[0]
                es = [jnp.full((16,), 1.0, jnp.float32)]
                for k in range(1, TOPK):
                    es.append(jnp.exp(vs[k] - m))
                s = es[0]
                for k in range(1, TOPK):
                    s = s + es[k]
                r = 1.0 / s
                for k in range(TOPK):
                    idxw_v[k, pl.ds(base + t0, 16)] = ixs[k]
                    wgtw_v[k, pl.ds(base + t0, 16)] = es[k] * r
                return carry

            return group

        cp_a.wait()
        lax.fori_loop(0, half // 16, make_group(lg_a, 0), 0)
        cp_b.wait()
        lax.fori_loop(0, half // 16, make_group(lg_b, half), 0)
        pltpu.sync_copy(idxw_v, idx_hbm.at[wid])
        pltpu.sync_copy(wgtw_v, wgt_hbm.at[wid])

    return _sc_topk_body


def _sc_topk_chunk(logits3, tpc, t):
    mesh = plsc.VectorSubcoreMesh(core_axis_name="c", subcore_axis_name="s")
    half = tpc // 2
    fn = pl.kernel(
        _make_sc_topk_body(tpc, t),
        out_type=[
            jax.ShapeDtypeStruct((NW, TOPK, tpc), jnp.int32),
            jax.ShapeDtypeStruct((NW, TOPK, tpc), jnp.float32),
        ],
        mesh=mesh,
        scratch_types=[
            pltpu.VMEM((NE, half), jnp.float32),
            pltpu.VMEM((NE, half), jnp.float32),
            pltpu.VMEM((TOPK, tpc), jnp.int32),
            pltpu.VMEM((TOPK, tpc), jnp.float32),
            pltpu.SemaphoreType.DMA,
            pltpu.SemaphoreType.DMA,
        ],
        cost_estimate=pl.CostEstimate(
            flops=50 * NW * NE * tpc,
            transcendentals=NW * TOPK * tpc,
            bytes_accessed=NW * NE * tpc * 4 + 2 * NW * TOPK * tpc * 4,
        ),
    )
    return fn(logits3)


def kernel(hidden_states, weight):
    x = hidden_states.reshape(-1, hidden_states.shape[-1])
    t = x.shape[0]
    tpc = t // (CHUNKS * NW)
    logits3 = _tc_logits_chunk(x, weight, tpc, 0)
    idx3, wgt3 = _sc_topk_chunk(logits3, tpc, t)
    idx = idx3.transpose(0, 2, 1).reshape(t, TOPK)
    wgt = wgt3.transpose(0, 2, 1).reshape(t, TOPK)
    return idx, wgt


# speedup vs baseline: 1.3014x; 1.2798x over previous
import functools

import jax
import jax.numpy as jnp
from jax import lax
from jax.experimental import pallas as pl
from jax.experimental.pallas import tpu as pltpu
from jax.experimental.pallas import tpu_sc as plsc

TOPK = 8
NE = 64
H = 1024
NC = 2
NS = 16
NW = NC * NS
CHUNKS = 1


TC_FAN = 2


def _logits_kernel(x_ref, w_ref, out_ref):
    w = w_ref[...]
    half = out_ref.shape[2]
    for j in range(out_ref.shape[0]):
        out_ref[j] = lax.dot_general(
            w, x_ref[pl.ds(j * half, half), :], (((1,), (1,)), ((), ())),
            preferred_element_type=jnp.float32,
        )


def _tc_logits_chunk(x, weight, tpc, chunk):
    half = tpc // 2
    nslab = 2 * TC_FAN
    return pl.pallas_call(
        _logits_kernel,
        grid=(NW // TC_FAN,),
        in_specs=[
            pl.BlockSpec((TC_FAN * tpc, H), lambda i, c=chunk: (c * NW // TC_FAN + i, 0)),
            pl.BlockSpec((NE, H), lambda i: (0, 0)),
        ],
        out_specs=pl.BlockSpec((nslab, NE, half), lambda i: (i, 0, 0)),
        out_shape=jax.ShapeDtypeStruct((2 * NW, NE, half), jnp.float32),
        cost_estimate=pl.CostEstimate(
            flops=2 * NW * tpc * H * NE,
            transcendentals=0,
            bytes_accessed=NW * tpc * H * 4 + NE * H * 4 + NW * NE * tpc * 4,
        ),
    )(x, weight)


SEL_BT = 2048


def _fused_kernel(x_ref, w_ref, idx_ref, wgt_ref):
    w = w_ref[...]
    lt = lax.dot_general(
        w, x_ref[...], (((1,), (1,)), ((), ())),
        preferred_element_type=jnp.float32,
    )
    iota = lax.broadcasted_iota(jnp.int32, lt.shape, 0)
    work = lt
    vals = []
    idxs = []
    for _ in range(TOPK):
        mk = jnp.max(work, axis=0, keepdims=True)
        cand = jnp.where(work == mk, iota, NE)
        ik = jnp.min(cand, axis=0, keepdims=True)
        vals.append(mk)
        idxs.append(ik)
        work = jnp.where(cand == ik, -jnp.inf, work)
    v = jnp.concatenate(vals, axis=0)
    e = jnp.exp(v - vals[0])
    s = jnp.sum(e, axis=0, keepdims=True)
    idx_ref[0] = jnp.concatenate(idxs, axis=0)
    wgt_ref[0] = e / s


def _tc_fused_chunk(x, weight, nsteps, step0):
    return pl.pallas_call(
        _fused_kernel,
        grid=(nsteps,),
        in_specs=[
            pl.BlockSpec((SEL_BT, H), lambda i, s=step0: (s + i, 0)),
            pl.BlockSpec((NE, H), lambda i: (0, 0)),
        ],
        out_specs=[
            pl.BlockSpec((1, TOPK, SEL_BT), lambda i: (i, 0, 0)),
            pl.BlockSpec((1, TOPK, SEL_BT), lambda i: (i, 0, 0)),
        ],
        out_shape=[
            jax.ShapeDtypeStruct((nsteps, TOPK, SEL_BT), jnp.int32),
            jax.ShapeDtypeStruct((nsteps, TOPK, SEL_BT), jnp.float32),
        ],
    )(x, weight)


def _make_sc_topk_body(tpc, t):
    half = tpc // 2

    def _sc_topk_body(lg_hbm, idx_hbm, wgt_hbm, lg_a, lg_b, idxw_v, wgtw_v,
                      sem_a, sem_b):
        wid = lax.axis_index("s") * NC + lax.axis_index("c")
        cp_a = pltpu.make_async_copy(lg_hbm.at[2 * wid], lg_a, sem_a)
        cp_b = pltpu.make_async_copy(lg_hbm.at[2 * wid + 1], lg_b, sem_b)
        cp_a.start()
        cp_b.start()

        def make_group(lg_v, base):
            def group(g, carry):
                t0 = pl.multiple_of(g * 16, 16)
                neg = jnp.full((16,), -jnp.inf, jnp.float32)
                zero = jnp.zeros((16,), jnp.int32)
                vs = [neg] * TOPK
                ixs = [zero] * TOPK
                for e in range(NE):
                    x = lg_v[e, pl.ds(t0, 16)]
                    ev = jnp.full((16,), e, jnp.int32)
                    gt = [x > vs[j] for j in range(TOPK)]
                    nv = [None] * TOPK
                    ni = [None] * TOPK
                    nv[0] = jnp.maximum(x, vs[0])
                    ni[0] = jnp.where(gt[0], ev, ixs[0])
                    for j in range(1, TOPK):
                        nv[j] = jnp.maximum(jnp.minimum(x, vs[j - 1]), vs[j])
                        ni[j] = jnp.where(gt[j], jnp.where(gt[j - 1], ixs[j - 1], ev), ixs[j])
                    vs, ixs = nv, ni
                m = vs[0]
                es = [jnp.full((16,), 1.0, jnp.float32)]
                for k in range(1, TOPK):
                    es.append(jnp.exp(vs[k] - m))
                s = es[0]
                for k in range(1, TOPK):
                    s = s + es[k]
                r = 1.0 / s
                for k in range(TOPK):
                    idxw_v[k, pl.ds(base + t0, 16)] = ixs[k]
                    wgtw_v[k, pl.ds(base + t0, 16)] = es[k] * r
                return carry

            return group

        cp_a.wait()
        lax.fori_loop(0, half // 16, make_group(lg_a, 0), 0)
        cp_b.wait()
        lax.fori_loop(0, half // 16, make_group(lg_b, half), 0)
        pltpu.sync_copy(idxw_v, idx_hbm.at[wid])
        pltpu.sync_copy(wgtw_v, wgt_hbm.at[wid])

    return _sc_topk_body


def _sc_topk_chunk(logits3, tpc, t):
    mesh = plsc.VectorSubcoreMesh(core_axis_name="c", subcore_axis_name="s")
    half = tpc // 2
    fn = pl.kernel(
        _make_sc_topk_body(tpc, t),
        out_type=[
            jax.ShapeDtypeStruct((NW, TOPK, tpc), jnp.int32),
            jax.ShapeDtypeStruct((NW, TOPK, tpc), jnp.float32),
        ],
        mesh=mesh,
        scratch_types=[
            pltpu.VMEM((NE, half), jnp.float32),
            pltpu.VMEM((NE, half), jnp.float32),
            pltpu.VMEM((TOPK, tpc), jnp.int32),
            pltpu.VMEM((TOPK, tpc), jnp.float32),
            pltpu.SemaphoreType.DMA,
            pltpu.SemaphoreType.DMA,
        ],
        cost_estimate=pl.CostEstimate(
            flops=50 * NW * NE * tpc,
            transcendentals=NW * TOPK * tpc,
            bytes_accessed=NW * NE * tpc * 4 + 2 * NW * TOPK * tpc * 4,
        ),
    )
    return fn(logits3)


SC_FRAC_NUM = 1
SC_FRAC_DEN = 2


def kernel(hidden_states, weight):
    x = hidden_states.reshape(-1, hidden_states.shape[-1])
    t = x.shape[0]
    sc_t = (t * SC_FRAC_NUM // SC_FRAC_DEN) // (NW * 16) * (NW * 16)
    tpc = sc_t // NW
    logits3 = _tc_logits_chunk(x, weight, tpc, 0)
    nsteps = (t - sc_t) // SEL_BT
    tidx, twgt = _tc_fused_chunk(x, weight, nsteps, sc_t // SEL_BT)
    idx3, wgt3 = _sc_topk_chunk(logits3, tpc, sc_t)
    sc_idx = idx3.transpose(0, 2, 1).reshape(sc_t, TOPK)
    sc_wgt = wgt3.transpose(0, 2, 1).reshape(sc_t, TOPK)
    tc_idx = tidx.transpose(0, 2, 1).reshape(t - sc_t, TOPK)
    tc_wgt = twgt.transpose(0, 2, 1).reshape(t - sc_t, TOPK)
    idx = jnp.concatenate([sc_idx, tc_idx], axis=0)
    wgt = jnp.concatenate([sc_wgt, tc_wgt], axis=0)
    return idx, wgt
